# baseline (speedup 1.0000x reference)
import functools

import jax
import jax.numpy as jnp
from jax import lax
from jax.experimental import pallas as pl
from jax.experimental.pallas import tpu as pltpu
from jax.experimental.pallas import tpu_sc as plsc

B = 16384
D = 64
NC = 2
NS = 16
L = 16
NW = NC * NS
BPW = B // NW
CHUNK = 128
NCHUNK = BPW // CHUNK


def _sc_body(user_hbm, item_hbm, ut_hbm, it_hbm, out_hbm,
             uidx_v, iidx_v, urows_v, irows_v, out_v, scr_v, sem_u, sem_i):
    wid = lax.axis_index("s") * NC + lax.axis_index("c")
    base = wid * BPW

    pltpu.sync_copy(user_hbm.at[wid], uidx_v)
    pltpu.sync_copy(item_hbm.at[wid], iidx_v)

    copies = []
    for j in range(NCHUNK):
        copies.append(pltpu.async_copy(
            ut_hbm.at[uidx_v.at[j]], urows_v.at[pl.ds(j * CHUNK, CHUNK)],
            sem_u))
        copies.append(pltpu.async_copy(
            it_hbm.at[iidx_v.at[j]], irows_v.at[pl.ds(j * CHUNK, CHUNK)],
            sem_i))
    for c in copies:
        c.wait()

    lanes = lax.iota(jnp.int32, L)

    def group_body(g, _):
        rbase = g * L
        for r16 in range(L):
            r = rbase + r16
            acc = jnp.zeros((L,), jnp.float32)
            for c in range(D // L):
                u = urows_v[r, pl.ds(c * L, L)]
                v = irows_v[r, pl.ds(c * L, L)]
                acc = acc + u * v
            scr_v[pl.ds(r16 * L, L)] = acc
        x = plsc.load_gather(scr_v, [lanes * L])
        for c in range(1, L):
            x = x + plsc.load_gather(scr_v, [lanes * L + c])
        e = jnp.exp(-jnp.abs(x))
        out_v[pl.ds(rbase, L)] = jnp.where(
            x >= 0, 1.0 / (1.0 + e), e / (1.0 + e))
        return 0

    lax.fori_loop(0, BPW // L, group_body, 0)

    pltpu.sync_copy(out_v, out_hbm.at[pl.ds(base, BPW)])


def kernel(user, item, user_table, item_table):
    mesh = plsc.VectorSubcoreMesh(core_axis_name="c", subcore_axis_name="s")
    k = functools.partial(
        pl.kernel,
        mesh=mesh,
        compiler_params=pltpu.CompilerParams(
            needs_layout_passes=False, use_tc_tiling_on_sc=False),
        out_type=jax.ShapeDtypeStruct((B,), jnp.float32),
        scratch_types=[
            pltpu.VMEM((NCHUNK, CHUNK), jnp.int32),
            pltpu.VMEM((NCHUNK, CHUNK), jnp.int32),
            pltpu.VMEM((BPW, D), jnp.float32),
            pltpu.VMEM((BPW, D), jnp.float32),
            pltpu.VMEM((BPW,), jnp.float32),
            pltpu.VMEM((L * L,), jnp.float32),
            pltpu.SemaphoreType.DMA,
            pltpu.SemaphoreType.DMA,
        ],
    )(_sc_body)
    return k(user.reshape(NW, NCHUNK, CHUNK), item.reshape(NW, NCHUNK, CHUNK),
             user_table, item_table)


# speedup vs baseline: 1.5450x; 1.5450x over previous
import functools

import jax
import jax.numpy as jnp
from jax import lax
from jax.experimental import pallas as pl
from jax.experimental.pallas import tpu as pltpu
from jax.experimental.pallas import tpu_sc as plsc

B = 16384
D = 64
NC = 2
NS = 16
L = 16
NW = NC * NS
BPW = B // NW
HALF = BPW // 2
NG = HALF // L


def _sc_body(user_hbm, item_hbm, ut_hbm, it_hbm, out_hbm,
             uidx_v, iidx_v, urows_v, irows_v, out_v, scr_v, sem_u, sem_i):
    wid = lax.axis_index("s") * NC + lax.axis_index("c")
    base = wid * BPW

    pltpu.sync_copy(user_hbm.at[pl.ds(base, BPW)], uidx_v)
    pltpu.sync_copy(item_hbm.at[pl.ds(base, BPW)], iidx_v)

    lanes = lax.iota(jnp.int32, L)

    def group_body(half, g, _):
        rbase = half * HALF + g * L
        vbase = g * L
        ivu = uidx_v[pl.ds(rbase, L)]
        ivi = iidx_v[pl.ds(rbase, L)]
        copies = []
        for j in range(L):
            copies.append(pltpu.async_copy(
                ut_hbm.at[ivu[j]], urows_v.at[vbase + j], sem_u))
            copies.append(pltpu.async_copy(
                it_hbm.at[ivi[j]], irows_v.at[vbase + j], sem_i))
        for c in copies:
            c.wait()

        for r16 in range(L):
            r = vbase + r16
            acc = urows_v[r, pl.ds(0, L)] * irows_v[r, pl.ds(0, L)]
            for c in range(1, D // L):
                acc = acc + (urows_v[r, pl.ds(c * L, L)]
                             * irows_v[r, pl.ds(c * L, L)])
            scr_v[pl.ds(r16 * L, L)] = acc
        x = plsc.load_gather(scr_v, [lanes * L])
        for c in range(1, L):
            x = x + plsc.load_gather(scr_v, [lanes * L + c])

        e = jnp.exp(-jnp.abs(x))
        out_v[pl.ds(rbase, L)] = jnp.where(
            x >= 0, 1.0 / (1.0 + e), e / (1.0 + e))
        return 0

    lax.fori_loop(0, NG, functools.partial(group_body, 0), 0)
    lax.fori_loop(0, NG, functools.partial(group_body, 1), 0)

    pltpu.sync_copy(out_v, out_hbm.at[pl.ds(base, BPW)])


def kernel(user, item, user_table, item_table):
    mesh = plsc.VectorSubcoreMesh(core_axis_name="c", subcore_axis_name="s")
    k = functools.partial(
        pl.kernel,
        mesh=mesh,
        compiler_params=pltpu.CompilerParams(
            needs_layout_passes=False, use_tc_tiling_on_sc=True),
        out_type=jax.ShapeDtypeStruct((B,), jnp.float32),
        scratch_types=[
            pltpu.VMEM((BPW,), jnp.int32),
            pltpu.VMEM((BPW,), jnp.int32),
            pltpu.VMEM((HALF, D), jnp.float32),
            pltpu.VMEM((HALF, D), jnp.float32),
            pltpu.VMEM((BPW,), jnp.float32),
            pltpu.VMEM((L * L,), jnp.float32),
            pltpu.SemaphoreType.DMA,
            pltpu.SemaphoreType.DMA,
        ],
    )(_sc_body)
    return k(user, item, user_table, item_table)
